# baseline (speedup 1.0000x reference)
import functools

import jax
import jax.numpy as jnp
from jax import lax
from jax.experimental import pallas as pl
from jax.experimental.pallas import tpu as pltpu
from jax.experimental.pallas import tpu_sc as plsc

B = 4096
K = 1024
D = 64
BETA = 0.25

BLK = 1024
NBLK = B // BLK

NC, NS = 1, 16
NW = NC * NS
BPW = B // NW


def _dist2(zb, cb):
    dot2 = lax.dot_general(zb + zb, cb, (((1,), (1,)), ((), ())),
                           preferred_element_type=jnp.float32)
    z2 = jnp.sum(zb * zb, axis=1, keepdims=True)
    c2 = jnp.sum(cb * cb, axis=1)
    return (z2 + c2[None, :]) - dot2


def _tc1_body(z_ref, cb_ref, idx_ref, loss_ref, loss_acc):
    i = pl.program_id(0)
    zb = z_ref[...]
    cb = cb_ref[...]
    d = _dist2(zb, cb)

    m = jnp.max(d, axis=1, keepdims=True)
    maskf = (d == m).astype(jnp.float32)
    j = lax.broadcasted_iota(jnp.int32, (K, 2), 0)
    ci = lax.broadcasted_iota(jnp.int32, (K, 2), 1)
    hilo = jnp.where(ci == 0, j >> 7, j & 127).astype(jnp.float32)
    idx2 = lax.dot_general(maskf, hilo, (((1,), (0,)), ((), ())),
                           preferred_element_type=jnp.float32)
    idx_f = idx2[:, 0:1] * 128.0 + idx2[:, 1:2]
    idx_ref[...] = idx_f.astype(jnp.int32)

    part = jnp.sum(m)

    @pl.when(i == 0)
    def _init():
        loss_acc[0] = 0.0

    loss_acc[0] += part

    @pl.when(i == NBLK - 1)
    def _fin():
        loss_ref[...] = (loss_acc[0] * ((1.0 + BETA) / (B * D))).reshape(1, 1)


def _tc2_body(z_ref, cb_ref, q_ref):
    zb = z_ref[...]
    cb = cb_ref[...]
    d = _dist2(zb, cb)
    qun = 1.0 / (1.0 + d)
    sinv = 1.0 / jnp.sum(qun, axis=1, keepdims=True)
    q_ref[...] = qun * sinv


_tc1_call = pl.pallas_call(
    _tc1_body,
    grid=(NBLK,),
    in_specs=[
        pl.BlockSpec((BLK, D), lambda i: (i, 0)),
        pl.BlockSpec((K, D), lambda i: (0, 0)),
    ],
    out_specs=[
        pl.BlockSpec((BLK, 1), lambda i: (i, 0)),
        pl.BlockSpec((1, 1), lambda i: (0, 0)),
    ],
    out_shape=[
        jax.ShapeDtypeStruct((B, 1), jnp.int32),
        jax.ShapeDtypeStruct((1, 1), jnp.float32),
    ],
    scratch_shapes=[pltpu.SMEM((1,), jnp.float32)],
)

_tc2_call = pl.pallas_call(
    _tc2_body,
    grid=(NBLK,),
    in_specs=[
        pl.BlockSpec((BLK, D), lambda i: (i, 0)),
        pl.BlockSpec((K, D), lambda i: (0, 0)),
    ],
    out_specs=pl.BlockSpec((BLK, K), lambda i: (i, 0)),
    out_shape=jax.ShapeDtypeStruct((B, K), jnp.float32),
)


def _sc_gather_body(cb_hbm, idx_hbm, zq_hbm, idx_v, rows_v, sem):
    wid = lax.axis_index("s") * NC + lax.axis_index("c")
    base = wid * BPW
    pltpu.sync_copy(idx_hbm.at[pl.ds(base, BPW)], idx_v)
    pltpu.async_copy(cb_hbm.at[idx_v], rows_v, sem).wait()
    pltpu.sync_copy(rows_v, zq_hbm.at[pl.ds(base, BPW)])


@functools.lru_cache(maxsize=None)
def _sc_gather_call():
    return pl.kernel(
        _sc_gather_body,
        out_type=jax.ShapeDtypeStruct((B, D), jnp.float32),
        mesh=plsc.VectorSubcoreMesh(core_axis_name="c", subcore_axis_name="s",
                                    num_cores=NC, num_subcores=NS),
        scratch_types=[
            pltpu.VMEM((BPW,), jnp.int32),
            pltpu.VMEM((BPW, D), jnp.float32),
            pltpu.SemaphoreType.DMA,
        ],
        compiler_params=pltpu.CompilerParams(use_tc_tiling_on_sc=False,
                                             skip_device_barrier=True),
    )


def kernel(z, codebook):
    idx2d, loss11 = _tc1_call(z, codebook)
    z_q = _sc_gather_call()(codebook, idx2d.reshape(B))
    q = _tc2_call(z, codebook)
    loss = loss11.reshape(())
    return (q, z_q, loss)


# speedup vs baseline: 1.0222x; 1.0222x over previous
import functools

import jax
import jax.numpy as jnp
from jax import lax
from jax.experimental import pallas as pl
from jax.experimental.pallas import tpu as pltpu
from jax.experimental.pallas import tpu_sc as plsc

B = 4096
K = 1024
D = 64
BETA = 0.25

BLK = 1024
NBLK = B // BLK

NC, NS = 2, 16
NW = NC * NS
BPW = B // NW


def _tc_body(z_ref, cb_ref, q_ref, idx_ref, loss_ref, loss_acc):
    i = pl.program_id(0)
    zb = z_ref[...]
    cb = cb_ref[...]

    dot2 = lax.dot_general(zb + zb, cb, (((1,), (1,)), ((), ())),
                           preferred_element_type=jnp.float32)
    z2 = jnp.sum(zb * zb, axis=1, keepdims=True)
    c2 = jnp.sum(cb * cb, axis=1)
    d = (z2 + c2[None, :]) - dot2

    qun = 1.0 / (1.0 + d)
    sinv = 1.0 / jnp.sum(qun, axis=1, keepdims=True)
    q_ref[...] = qun * sinv

    m = jnp.max(d, axis=1, keepdims=True)
    maskf = (d == m).astype(jnp.float32)
    j = lax.broadcasted_iota(jnp.int32, (K, 2), 0)
    ci = lax.broadcasted_iota(jnp.int32, (K, 2), 1)
    hilo = jnp.where(ci == 0, j >> 7, j & 127).astype(jnp.float32)
    idx2 = lax.dot_general(maskf, hilo, (((1,), (0,)), ((), ())),
                           preferred_element_type=jnp.float32)
    idx_f = idx2[:, 0:1] * 128.0 + idx2[:, 1:2]
    idx_ref[...] = idx_f.astype(jnp.int32)

    part = jnp.sum(m)

    @pl.when(i == 0)
    def _init():
        loss_acc[0] = 0.0

    loss_acc[0] += part

    @pl.when(i == NBLK - 1)
    def _fin():
        loss_ref[...] = (loss_acc[0] * ((1.0 + BETA) / (B * D))).reshape(1, 1)


_tc_call = pl.pallas_call(
    _tc_body,
    grid=(NBLK,),
    in_specs=[
        pl.BlockSpec((BLK, D), lambda i: (i, 0)),
        pl.BlockSpec((K, D), lambda i: (0, 0)),
    ],
    out_specs=[
        pl.BlockSpec((BLK, K), lambda i: (i, 0)),
        pl.BlockSpec((BLK, 1), lambda i: (i, 0)),
        pl.BlockSpec((1, 1), lambda i: (0, 0)),
    ],
    out_shape=[
        jax.ShapeDtypeStruct((B, K), jnp.float32),
        jax.ShapeDtypeStruct((B, 1), jnp.int32),
        jax.ShapeDtypeStruct((1, 1), jnp.float32),
    ],
    scratch_shapes=[pltpu.SMEM((1,), jnp.float32)],
)


def _sc_gather_body(cb_hbm, idx_hbm, zq_hbm, idx_v, rows_v, sem):
    wid = lax.axis_index("s") * NC + lax.axis_index("c")
    base = wid * BPW
    pltpu.sync_copy(idx_hbm.at[pl.ds(base, BPW)], idx_v)
    pltpu.async_copy(cb_hbm.at[idx_v], rows_v, sem).wait()
    pltpu.sync_copy(rows_v, zq_hbm.at[pl.ds(base, BPW)])


@functools.lru_cache(maxsize=None)
def _sc_gather_call():
    return pl.kernel(
        _sc_gather_body,
        out_type=jax.ShapeDtypeStruct((B, D), jnp.float32),
        mesh=plsc.VectorSubcoreMesh(core_axis_name="c", subcore_axis_name="s",
                                    num_cores=NC, num_subcores=NS),
        scratch_types=[
            pltpu.VMEM((BPW,), jnp.int32),
            pltpu.VMEM((BPW, D), jnp.float32),
            pltpu.SemaphoreType.DMA,
        ],
        compiler_params=pltpu.CompilerParams(use_tc_tiling_on_sc=False,
                                             skip_device_barrier=True),
    )


def kernel(z, codebook):
    q, idx2d, loss11 = _tc_call(z, codebook)
    z_q = _sc_gather_call()(codebook, idx2d.reshape(B))
    loss = loss11.reshape(())
    return (q, z_q, loss)
